# baseline (speedup 1.0000x reference)
import functools

import jax
import jax.numpy as jnp
from jax import lax
from jax.experimental import pallas as pl
from jax.experimental.pallas import tpu as pltpu
from jax.experimental.pallas import tpu_sc as plsc


@functools.cache
def _make_gather(V4, D, NB_ROWS, NB_COLS):
    info = plsc.get_sparse_core_info()
    NC, NS = info.num_cores, info.num_subcores
    NW = NC * NS
    assert NB_ROWS % NW == 0
    W = NB_ROWS // NW
    n_tasks = NB_COLS
    mesh = plsc.VectorSubcoreMesh(core_axis_name="c", subcore_axis_name="s")

    @functools.partial(
        pl.kernel,
        mesh=mesh,
        compiler_params=pltpu.CompilerParams(
            use_tc_tiling_on_sc=False, needs_layout_passes=False
        ),
        out_type=jax.ShapeDtypeStruct((NB_COLS, D, NB_ROWS), jnp.float32),
        scratch_types=[
            [pltpu.VMEM((W,), jnp.int32) for _ in range(2)],
            [pltpu.VMEM((W, D), jnp.float32) for _ in range(2)],
            [pltpu.VMEM((D, W), jnp.float32) for _ in range(2)],
            [pltpu.SemaphoreType.DMA for _ in range(2)],
            [pltpu.SemaphoreType.DMA for _ in range(2)],
        ],
    )
    def gather_kernel(table_hbm, idx_hbm, out_hbm, idxv, rows, outT, gsem, wsem):
        wid = lax.axis_index("s") * NC + lax.axis_index("c")
        b0 = wid * W
        iota = lax.iota(jnp.int32, 16)

        def start_gather(i):
            buf = i % 2
            pltpu.sync_copy(idx_hbm.at[pl.ds(i * NB_ROWS + b0, W)], idxv[buf])
            return pltpu.async_copy(table_hbm.at[idxv[buf]], rows[buf], gsem[buf])

        def transpose_task(buf):
            def body(d, carry):
                col = jnp.full((16,), d, jnp.int32)
                for jg in range(W // 16):
                    v = plsc.load_gather(rows[buf], [iota + (jg * 16), col])
                    outT[buf][d, pl.ds(jg * 16, 16)] = v
                return carry

            lax.fori_loop(0, D, body, 0)

        gcopy = [None] * n_tasks
        wcopy = [None] * n_tasks
        gcopy[0] = start_gather(0)
        for i in range(n_tasks):
            buf = i % 2
            if i + 1 < n_tasks:
                gcopy[i + 1] = start_gather(i + 1)
            gcopy[i].wait()
            if i >= 2:
                wcopy[i - 2].wait()
            transpose_task(buf)
            wcopy[i] = pltpu.async_copy(
                outT[buf], out_hbm.at[i, :, pl.ds(b0, W)], wsem[buf]
            )
        for i in range(max(0, n_tasks - 2), n_tasks):
            wcopy[i].wait()

    return gather_kernel


def kernel(X, weight):
    rows, cols = X.shape
    V, D = weight.shape
    flat_idx = X.T.reshape(rows * cols).astype(jnp.int32) * 4
    wview = jnp.pad(weight, ((0, 0), (0, 96))).reshape(V * 4, D)
    out = _make_gather(V * 4, D, rows, cols)(wview, flat_idx)
    return out.transpose(2, 0, 1)


# speedup vs baseline: 1.3285x; 1.3285x over previous
import functools

import jax
import jax.numpy as jnp
from jax import lax
from jax.experimental import pallas as pl
from jax.experimental.pallas import tpu as pltpu
from jax.experimental.pallas import tpu_sc as plsc


@functools.cache
def _make_gather(V4, D, NB_ROWS, NB_COLS):
    info = plsc.get_sparse_core_info()
    NC, NS = info.num_cores, info.num_subcores
    NW = NC * NS
    assert NB_ROWS % NW == 0
    W = NB_ROWS // NW
    n_tasks = NB_COLS
    mesh = plsc.VectorSubcoreMesh(core_axis_name="c", subcore_axis_name="s")

    @functools.partial(
        pl.kernel,
        mesh=mesh,
        compiler_params=pltpu.CompilerParams(
            use_tc_tiling_on_sc=False, needs_layout_passes=False
        ),
        out_type=jax.ShapeDtypeStruct((NB_COLS, D, NB_ROWS), jnp.float32),
        scratch_types=[
            [pltpu.VMEM((W,), jnp.int32) for _ in range(2)],
            [pltpu.VMEM((W, D), jnp.float32) for _ in range(2)],
            [pltpu.VMEM((D, W), jnp.float32) for _ in range(2)],
            [pltpu.SemaphoreType.DMA for _ in range(2)],
            [pltpu.SemaphoreType.DMA for _ in range(2)],
        ],
    )
    def gather_kernel(table_hbm, idx_hbm, out_hbm, idxv, rows, outT, gsem, wsem):
        wid = lax.axis_index("s") * NC + lax.axis_index("c")
        b0 = wid * W
        iota = lax.iota(jnp.int32, 16)

        def start_gather(i, buf):
            pltpu.sync_copy(idx_hbm.at[pl.ds(i * NB_ROWS + b0, W)], idxv[buf])
            pltpu.async_copy(table_hbm.at[idxv[buf]], rows[buf], gsem[buf])

        def wait_gather(buf):
            pltpu.make_async_copy(
                table_hbm.at[idxv[buf]], rows[buf], gsem[buf]
            ).wait()

        def start_write(i, buf):
            pltpu.async_copy(
                outT[buf], out_hbm.at[i, :, pl.ds(b0, W)], wsem[buf]
            )

        def wait_write(buf):
            pltpu.make_async_copy(
                outT[buf], out_hbm.at[0, :, pl.ds(b0, W)], wsem[buf]
            ).wait()

        sdiag = [(iota + k) & 15 for k in range(16)]

        def transpose_task(buf):
            def body(jb, carry):
                j0 = jb * 16
                for db in range(D // 16):
                    d0 = db * 16
                    for k in range(16):
                        v = plsc.load_gather(
                            rows[buf], [iota + j0, sdiag[k] + d0]
                        )
                        plsc.store_scatter(
                            outT[buf], [sdiag[k] + d0, iota + j0], v
                        )
                return carry

            lax.fori_loop(0, W // 16, body, 0)

        assert n_tasks % 2 == 0
        start_gather(0, 0)

        def pair_body(p, carry):
            for buf in range(2):
                i = 2 * p + buf

                @pl.when(i + 1 < n_tasks)
                def _():
                    start_gather(i + 1, 1 - buf)

                wait_gather(buf)

                @pl.when(i >= 2)
                def _():
                    wait_write(buf)

                transpose_task(buf)
                start_write(i, buf)
            return carry

        lax.fori_loop(0, n_tasks // 2, pair_body, 0)
        wait_write(0)
        wait_write(1)

    return gather_kernel


def kernel(X, weight):
    rows, cols = X.shape
    V, D = weight.shape
    flat_idx = X.T.reshape(rows * cols).astype(jnp.int32) * 4
    wview = jnp.pad(weight, ((0, 0), (0, 96))).reshape(V * 4, D)
    out = _make_gather(V * 4, D, rows, cols)(wview, flat_idx)
    return out.transpose(2, 0, 1)


# speedup vs baseline: 1.9458x; 1.4647x over previous
import functools

import jax
import jax.numpy as jnp
from jax import lax
from jax.experimental import pallas as pl
from jax.experimental.pallas import tpu as pltpu
from jax.experimental.pallas import tpu_sc as plsc


@functools.cache
def _make_format(V, D):
    info = plsc.get_sparse_core_info()
    NC, NS = info.num_cores, info.num_subcores
    NW = NC * NS
    TCOLS = V // 128
    R4 = V // 4
    base_cnt = TCOLS // NW
    rem = TCOLS % NW
    mesh = plsc.VectorSubcoreMesh(core_axis_name="c", subcore_axis_name="s")

    @functools.partial(
        pl.kernel,
        mesh=mesh,
        compiler_params=pltpu.CompilerParams(
            use_tc_tiling_on_sc=True, needs_layout_passes=False
        ),
        out_type=jax.ShapeDtypeStruct((R4, 128), jnp.float32),
        scratch_types=[
            [pltpu.VMEM((D, 128), jnp.float32) for _ in range(2)],
            [pltpu.VMEM((32, 128), jnp.float32) for _ in range(2)],
            [pltpu.SemaphoreType.DMA for _ in range(2)],
            [pltpu.SemaphoreType.DMA for _ in range(2)],
        ],
    )
    def format_kernel(wt_hbm, tail_hbm, w4_hbm, S, O, lsem, wsem):
        wid = lax.axis_index("s") * NC + lax.axis_index("c")
        iota = lax.iota(jnp.int32, 16)
        nblk = base_cnt + jnp.where(wid < rem, 1, 0)
        start = wid * base_cnt + jnp.minimum(wid, rem)

        grow = [(iota + k) & 15 for k in range(16)]
        orow = iota // 4
        ocol = [32 * (iota % 4) + ((iota + k) & 15) for k in range(16)]

        def blk(j):
            return pl.multiple_of((start + j) * 128, 128)

        def start_load(j, buf):
            pltpu.async_copy(
                wt_hbm.at[:, pl.ds(blk(j), 128)], S[buf], lsem[buf]
            )

        def wait_load(buf):
            pltpu.make_async_copy(
                wt_hbm.at[:, pl.ds(0, 128)], S[buf], lsem[buf]
            ).wait()

        def start_write(j, buf):
            pltpu.async_copy(
                O[buf],
                w4_hbm.at[pl.ds(pl.multiple_of((start + j) * 32, 32), 32), :],
                wsem[buf],
            )

        def wait_write(buf):
            pltpu.make_async_copy(
                O[buf], w4_hbm.at[pl.ds(0, 32), :], wsem[buf]
            ).wait()

        def permute(buf):
            def pbody(m, carry):
                dh16 = (m // 8) * 16
                u0 = (m % 8) * 16
                for k in range(16):
                    v = plsc.load_gather(S[buf], [grow[k] + dh16, iota + u0])
                    plsc.store_scatter(
                        O[buf], [orow + (u0 // 4), ocol[k] + dh16], v
                    )
                return carry

            lax.fori_loop(0, 16, pbody, 0)

        start_load(0, 0)

        def pair_body(p, carry):
            for buf in range(2):
                j = 2 * p + buf

                @pl.when(j < nblk)
                def _(j=j, buf=buf):
                    @pl.when(j + 1 < nblk)
                    def _():
                        start_load(j + 1, 1 - buf)

                    wait_load(buf)

                    @pl.when(j >= 2)
                    def _():
                        wait_write(buf)

                    permute(buf)
                    start_write(j, buf)

            return carry

        lax.fori_loop(0, (base_cnt + 2) // 2, pair_body, 0)
        wait_write(0)
        wait_write(1)

        @pl.when(wid == NW - 1)
        def _():
            pltpu.sync_copy(tail_hbm, S[0].at[pl.ds(0, 16), :])
            pltpu.sync_copy(
                S[0].at[pl.ds(0, 16), :], w4_hbm.at[pl.ds(R4 - 16, 16), :]
            )

    return format_kernel


@functools.cache
def _make_gather(V4, D, NB_ROWS, NB_COLS):
    info = plsc.get_sparse_core_info()
    NC, NS = info.num_cores, info.num_subcores
    NW = NC * NS
    assert NB_ROWS % NW == 0
    W = NB_ROWS // NW
    n_tasks = NB_COLS
    mesh = plsc.VectorSubcoreMesh(core_axis_name="c", subcore_axis_name="s")

    @functools.partial(
        pl.kernel,
        mesh=mesh,
        compiler_params=pltpu.CompilerParams(
            use_tc_tiling_on_sc=False, needs_layout_passes=False
        ),
        out_type=jax.ShapeDtypeStruct((NB_COLS, D, NB_ROWS), jnp.float32),
        scratch_types=[
            [pltpu.VMEM((W,), jnp.int32) for _ in range(2)],
            [pltpu.VMEM((W, D), jnp.float32) for _ in range(2)],
            [pltpu.VMEM((D, W), jnp.float32) for _ in range(2)],
            [pltpu.SemaphoreType.DMA for _ in range(2)],
            [pltpu.SemaphoreType.DMA for _ in range(2)],
        ],
    )
    def gather_kernel(table_hbm, idx_hbm, out_hbm, idxv, rows, outT, gsem, wsem):
        wid = lax.axis_index("s") * NC + lax.axis_index("c")
        b0 = wid * W
        iota = lax.iota(jnp.int32, 16)

        def start_gather(i, buf):
            pltpu.sync_copy(idx_hbm.at[pl.ds(i * NB_ROWS + b0, W)], idxv[buf])
            pltpu.async_copy(table_hbm.at[idxv[buf]], rows[buf], gsem[buf])

        def wait_gather(buf):
            pltpu.make_async_copy(
                table_hbm.at[idxv[buf]], rows[buf], gsem[buf]
            ).wait()

        def start_write(i, buf):
            pltpu.async_copy(
                outT[buf], out_hbm.at[i, :, pl.ds(b0, W)], wsem[buf]
            )

        def wait_write(buf):
            pltpu.make_async_copy(
                outT[buf], out_hbm.at[0, :, pl.ds(b0, W)], wsem[buf]
            ).wait()

        sdiag = [(iota + k) & 15 for k in range(16)]

        def transpose_task(buf):
            def body(jb, carry):
                j0 = jb * 16
                for db in range(D // 16):
                    d0 = db * 16
                    for k in range(16):
                        v = plsc.load_gather(
                            rows[buf], [iota + j0, sdiag[k] + d0]
                        )
                        plsc.store_scatter(
                            outT[buf], [sdiag[k] + d0, iota + j0], v
                        )
                return carry

            lax.fori_loop(0, W // 16, body, 0)

        assert n_tasks % 2 == 0
        start_gather(0, 0)

        def pair_body(p, carry):
            for buf in range(2):
                i = 2 * p + buf

                @pl.when(i + 1 < n_tasks)
                def _():
                    start_gather(i + 1, 1 - buf)

                wait_gather(buf)

                @pl.when(i >= 2)
                def _():
                    wait_write(buf)

                transpose_task(buf)
                start_write(i, buf)
            return carry

        lax.fori_loop(0, n_tasks // 2, pair_body, 0)
        wait_write(0)
        wait_write(1)

    return gather_kernel


def kernel(X, weight):
    rows, cols = X.shape
    V, D = weight.shape
    flat_idx = X.T.reshape(rows * cols).astype(jnp.int32)
    tail = weight[V - 64 :].reshape(16, 128)
    w4 = _make_format(V, D)(weight.T, tail)
    out = _make_gather(V, D, rows, cols)(w4.reshape(V, D), flat_idx)
    return out.transpose(2, 0, 1)


# speedup vs baseline: 1.9626x; 1.0086x over previous
import functools

import jax
import jax.numpy as jnp
from jax import lax
from jax.experimental import pallas as pl
from jax.experimental.pallas import tpu as pltpu
from jax.experimental.pallas import tpu_sc as plsc


@functools.cache
def _make_format(V, D):
    info = plsc.get_sparse_core_info()
    NC, NS = info.num_cores, info.num_subcores
    NW = NC * NS
    TCOLS = V // 128
    R4 = V // 4
    base_cnt = TCOLS // NW
    rem = TCOLS % NW
    mesh = plsc.VectorSubcoreMesh(core_axis_name="c", subcore_axis_name="s")

    @functools.partial(
        pl.kernel,
        mesh=mesh,
        compiler_params=pltpu.CompilerParams(
            use_tc_tiling_on_sc=True, needs_layout_passes=False
        ),
        out_type=jax.ShapeDtypeStruct((R4, 128), jnp.float32),
        scratch_types=[
            [pltpu.VMEM((D, 128), jnp.float32) for _ in range(2)],
            [pltpu.VMEM((32, 128), jnp.float32) for _ in range(2)],
            [pltpu.SemaphoreType.DMA for _ in range(2)],
            [pltpu.SemaphoreType.DMA for _ in range(2)],
        ],
    )
    def format_kernel(wt_hbm, tail_hbm, w4_hbm, S, O, lsem, wsem):
        wid = lax.axis_index("s") * NC + lax.axis_index("c")
        iota = lax.iota(jnp.int32, 16)
        nblk = base_cnt + jnp.where(wid < rem, 1, 0)
        start = wid * base_cnt + jnp.minimum(wid, rem)

        grow = [(iota + k) & 15 for k in range(16)]
        orow = iota // 4
        ocol = [32 * (iota % 4) + ((iota + k) & 15) for k in range(16)]

        def blk(j):
            return pl.multiple_of((start + j) * 128, 128)

        def start_load(j, buf):
            pltpu.async_copy(
                wt_hbm.at[:, pl.ds(blk(j), 128)], S[buf], lsem[buf]
            )

        def wait_load(buf):
            pltpu.make_async_copy(
                wt_hbm.at[:, pl.ds(0, 128)], S[buf], lsem[buf]
            ).wait()

        def start_write(j, buf):
            pltpu.async_copy(
                O[buf],
                w4_hbm.at[pl.ds(pl.multiple_of((start + j) * 32, 32), 32), :],
                wsem[buf],
            )

        def wait_write(buf):
            pltpu.make_async_copy(
                O[buf], w4_hbm.at[pl.ds(0, 32), :], wsem[buf]
            ).wait()

        def permute(buf):
            def pbody(m, carry):
                for half in range(2):
                    mm = 2 * m + half
                    dh16 = (mm // 8) * 16
                    u0 = (mm % 8) * 16
                    for k in range(16):
                        v = plsc.load_gather(
                            S[buf], [grow[k] + dh16, iota + u0]
                        )
                        plsc.store_scatter(
                            O[buf], [orow + (u0 // 4), ocol[k] + dh16], v
                        )
                return carry

            lax.fori_loop(0, 8, pbody, 0)

        start_load(0, 0)

        def pair_body(p, carry):
            for buf in range(2):
                j = 2 * p + buf

                @pl.when(j < nblk)
                def _(j=j, buf=buf):
                    @pl.when(j + 1 < nblk)
                    def _():
                        start_load(j + 1, 1 - buf)

                    wait_load(buf)

                    @pl.when(j >= 2)
                    def _():
                        wait_write(buf)

                    permute(buf)
                    start_write(j, buf)

            return carry

        lax.fori_loop(0, (base_cnt + 2) // 2, pair_body, 0)
        wait_write(0)
        wait_write(1)

        @pl.when(wid == NW - 1)
        def _():
            pltpu.sync_copy(tail_hbm, S[0].at[pl.ds(0, 16), :])
            pltpu.sync_copy(
                S[0].at[pl.ds(0, 16), :], w4_hbm.at[pl.ds(R4 - 16, 16), :]
            )

    return format_kernel


@functools.cache
def _make_gather(V4, D, NB_ROWS, NB_COLS):
    info = plsc.get_sparse_core_info()
    NC, NS = info.num_cores, info.num_subcores
    NW = NC * NS
    assert NB_ROWS % NW == 0
    W = NB_ROWS // NW
    n_tasks = NB_COLS
    mesh = plsc.VectorSubcoreMesh(core_axis_name="c", subcore_axis_name="s")

    @functools.partial(
        pl.kernel,
        mesh=mesh,
        compiler_params=pltpu.CompilerParams(
            use_tc_tiling_on_sc=False, needs_layout_passes=False
        ),
        out_type=jax.ShapeDtypeStruct((NB_COLS, D, NB_ROWS), jnp.float32),
        scratch_types=[
            [pltpu.VMEM((W,), jnp.int32) for _ in range(2)],
            [pltpu.VMEM((W, D), jnp.float32) for _ in range(2)],
            [pltpu.VMEM((D, W), jnp.float32) for _ in range(2)],
            [pltpu.SemaphoreType.DMA for _ in range(2)],
            [pltpu.SemaphoreType.DMA for _ in range(2)],
        ],
    )
    def gather_kernel(table_hbm, idx_hbm, out_hbm, idxv, rows, outT, gsem, wsem):
        wid = lax.axis_index("s") * NC + lax.axis_index("c")
        b0 = wid * W
        iota = lax.iota(jnp.int32, 16)

        def start_gather(i, buf):
            pltpu.sync_copy(idx_hbm.at[pl.ds(i * NB_ROWS + b0, W)], idxv[buf])
            pltpu.async_copy(table_hbm.at[idxv[buf]], rows[buf], gsem[buf])

        def wait_gather(buf):
            pltpu.make_async_copy(
                table_hbm.at[idxv[buf]], rows[buf], gsem[buf]
            ).wait()

        def start_write(i, buf):
            pltpu.async_copy(
                outT[buf], out_hbm.at[i, :, pl.ds(b0, W)], wsem[buf]
            )

        def wait_write(buf):
            pltpu.make_async_copy(
                outT[buf], out_hbm.at[0, :, pl.ds(b0, W)], wsem[buf]
            ).wait()

        sdiag = [(iota + k) & 15 for k in range(16)]

        def transpose_task(buf):
            def body(jb, carry):
                j0 = jb * 16
                for db in range(D // 16):
                    d0 = db * 16
                    for k in range(16):
                        v = plsc.load_gather(
                            rows[buf], [iota + j0, sdiag[k] + d0]
                        )
                        plsc.store_scatter(
                            outT[buf], [sdiag[k] + d0, iota + j0], v
                        )
                return carry

            lax.fori_loop(0, W // 16, body, 0)

        assert n_tasks % 2 == 0
        start_gather(0, 0)

        def pair_body(p, carry):
            for buf in range(2):
                i = 2 * p + buf

                @pl.when(i + 1 < n_tasks)
                def _():
                    start_gather(i + 1, 1 - buf)

                wait_gather(buf)

                @pl.when(i >= 2)
                def _():
                    wait_write(buf)

                transpose_task(buf)
                start_write(i, buf)
            return carry

        lax.fori_loop(0, n_tasks // 2, pair_body, 0)
        wait_write(0)
        wait_write(1)

    return gather_kernel


def kernel(X, weight):
    rows, cols = X.shape
    V, D = weight.shape
    flat_idx = X.T.reshape(rows * cols).astype(jnp.int32)
    tail = weight[V - 64 :].reshape(16, 128)
    w4 = _make_format(V, D)(weight.T, tail)
    out = _make_gather(V, D, rows, cols)(w4.reshape(V, D), flat_idx)
    return out.transpose(2, 0, 1)


# speedup vs baseline: 2.5594x; 1.3041x over previous
import functools

import jax
import jax.numpy as jnp
from jax import lax
from jax.experimental import pallas as pl
from jax.experimental.pallas import tpu as pltpu
from jax.experimental.pallas import tpu_sc as plsc


@functools.cache
def _make_format(V, D):
    info = plsc.get_sparse_core_info()
    NC, NS = info.num_cores, info.num_subcores
    NW = NC * NS
    TCOLS = V // 128
    R4 = V // 4
    base_cnt = TCOLS // NW
    rem = TCOLS % NW
    mesh = plsc.VectorSubcoreMesh(core_axis_name="c", subcore_axis_name="s")

    @functools.partial(
        pl.kernel,
        mesh=mesh,
        compiler_params=pltpu.CompilerParams(
            use_tc_tiling_on_sc=True, needs_layout_passes=False
        ),
        out_type=jax.ShapeDtypeStruct((R4, 128), jnp.float32),
        scratch_types=[
            [pltpu.VMEM((D, 128), jnp.float32) for _ in range(2)],
            [pltpu.VMEM((32, 128), jnp.float32) for _ in range(2)],
            [pltpu.SemaphoreType.DMA for _ in range(2)],
            [pltpu.SemaphoreType.DMA for _ in range(2)],
        ],
    )
    def format_kernel(wt_hbm, tail_hbm, w4_hbm, S, O, lsem, wsem):
        wid = lax.axis_index("s") * NC + lax.axis_index("c")
        iota = lax.iota(jnp.int32, 16)
        nblk = base_cnt + jnp.where(wid < rem, 1, 0)
        start = wid * base_cnt + jnp.minimum(wid, rem)

        grow = [(iota + k) & 15 for k in range(16)]
        orow = iota // 4
        ocol = [32 * (iota % 4) + ((iota + k) & 15) for k in range(16)]

        def blk(j):
            return pl.multiple_of((start + j) * 128, 128)

        def start_load(j, buf):
            pltpu.async_copy(
                wt_hbm.at[:, pl.ds(blk(j), 128)], S[buf], lsem[buf]
            )

        def wait_load(buf):
            pltpu.make_async_copy(
                wt_hbm.at[:, pl.ds(0, 128)], S[buf], lsem[buf]
            ).wait()

        def start_write(j, buf):
            pltpu.async_copy(
                O[buf],
                w4_hbm.at[pl.ds(pl.multiple_of((start + j) * 32, 32), 32), :],
                wsem[buf],
            )

        def wait_write(buf):
            pltpu.make_async_copy(
                O[buf], w4_hbm.at[pl.ds(0, 32), :], wsem[buf]
            ).wait()

        def permute(buf):
            def pbody(m, carry):
                for half in range(2):
                    mm = 2 * m + half
                    dh16 = (mm // 8) * 16
                    u0 = (mm % 8) * 16
                    vs = [
                        plsc.load_gather(S[buf], [grow[k] + dh16, iota + u0])
                        for k in range(16)
                    ]
                    for k in range(16):
                        plsc.store_scatter(
                            O[buf], [orow + (u0 // 4), ocol[k] + dh16], vs[k]
                        )
                return carry

            lax.fori_loop(0, 8, pbody, 0)

        start_load(0, 0)

        def pair_body(p, carry):
            for buf in range(2):
                j = 2 * p + buf

                @pl.when(j < nblk)
                def _(j=j, buf=buf):
                    @pl.when(j + 1 < nblk)
                    def _():
                        start_load(j + 1, 1 - buf)

                    wait_load(buf)

                    @pl.when(j >= 2)
                    def _():
                        wait_write(buf)

                    permute(buf)
                    start_write(j, buf)

            return carry

        lax.fori_loop(0, (base_cnt + 2) // 2, pair_body, 0)
        wait_write(0)
        wait_write(1)

        @pl.when(wid == NW - 1)
        def _():
            pltpu.sync_copy(tail_hbm, S[0].at[pl.ds(0, 16), :])
            pltpu.sync_copy(
                S[0].at[pl.ds(0, 16), :], w4_hbm.at[pl.ds(R4 - 16, 16), :]
            )

    return format_kernel


@functools.cache
def _make_gather(V4, D, NB_ROWS, NB_COLS):
    info = plsc.get_sparse_core_info()
    NC, NS = info.num_cores, info.num_subcores
    NW = NC * NS
    assert NB_ROWS % NW == 0
    W = NB_ROWS // NW
    n_tasks = NB_COLS
    mesh = plsc.VectorSubcoreMesh(core_axis_name="c", subcore_axis_name="s")

    @functools.partial(
        pl.kernel,
        mesh=mesh,
        compiler_params=pltpu.CompilerParams(
            use_tc_tiling_on_sc=False, needs_layout_passes=False
        ),
        out_type=jax.ShapeDtypeStruct((NB_COLS, D, NB_ROWS), jnp.float32),
        scratch_types=[
            [pltpu.VMEM((W,), jnp.int32) for _ in range(2)],
            [pltpu.VMEM((W, D), jnp.float32) for _ in range(2)],
            [pltpu.VMEM((D, W), jnp.float32) for _ in range(2)],
            [pltpu.SemaphoreType.DMA for _ in range(2)],
            [pltpu.SemaphoreType.DMA for _ in range(2)],
        ],
    )
    def gather_kernel(table_hbm, idx_hbm, out_hbm, idxv, rows, outT, gsem, wsem):
        wid = lax.axis_index("s") * NC + lax.axis_index("c")
        b0 = wid * W
        iota = lax.iota(jnp.int32, 16)

        def start_gather(i, buf):
            pltpu.sync_copy(idx_hbm.at[pl.ds(i * NB_ROWS + b0, W)], idxv[buf])
            pltpu.async_copy(table_hbm.at[idxv[buf]], rows[buf], gsem[buf])

        def wait_gather(buf):
            pltpu.make_async_copy(
                table_hbm.at[idxv[buf]], rows[buf], gsem[buf]
            ).wait()

        def start_write(i, buf):
            pltpu.async_copy(
                outT[buf], out_hbm.at[i, :, pl.ds(b0, W)], wsem[buf]
            )

        def wait_write(buf):
            pltpu.make_async_copy(
                outT[buf], out_hbm.at[0, :, pl.ds(b0, W)], wsem[buf]
            ).wait()

        sdiag = [(iota + k) & 15 for k in range(16)]

        def transpose_task(buf):
            def body(jb, carry):
                j0 = jb * 16
                for db in range(D // 16):
                    d0 = db * 16
                    vs = [
                        plsc.load_gather(rows[buf], [iota + j0, sdiag[k] + d0])
                        for k in range(16)
                    ]
                    for k in range(16):
                        plsc.store_scatter(
                            outT[buf], [sdiag[k] + d0, iota + j0], vs[k]
                        )
                return carry

            lax.fori_loop(0, W // 16, body, 0)

        assert n_tasks % 2 == 0
        start_gather(0, 0)

        def pair_body(p, carry):
            for buf in range(2):
                i = 2 * p + buf

                @pl.when(i + 1 < n_tasks)
                def _():
                    start_gather(i + 1, 1 - buf)

                wait_gather(buf)

                @pl.when(i >= 2)
                def _():
                    wait_write(buf)

                transpose_task(buf)
                start_write(i, buf)
            return carry

        lax.fori_loop(0, n_tasks // 2, pair_body, 0)
        wait_write(0)
        wait_write(1)

    return gather_kernel


def kernel(X, weight):
    rows, cols = X.shape
    V, D = weight.shape
    flat_idx = X.T.reshape(rows * cols).astype(jnp.int32)
    tail = weight[V - 64 :].reshape(16, 128)
    w4 = _make_format(V, D)(weight.T, tail)
    out = _make_gather(V, D, rows, cols)(w4.reshape(V, D), flat_idx)
    return out.transpose(2, 0, 1)


# speedup vs baseline: 2.9806x; 1.1646x over previous
import functools

import jax
import jax.numpy as jnp
from jax import lax
from jax.experimental import pallas as pl
from jax.experimental.pallas import tpu as pltpu
from jax.experimental.pallas import tpu_sc as plsc


@functools.cache
def _make_format(V, D):
    info = plsc.get_sparse_core_info()
    NC, NS = info.num_cores, info.num_subcores
    NW = NC * NS
    TCOLS = V // 256
    R4 = V // 4
    base_cnt = TCOLS // NW
    rem = TCOLS % NW
    mesh = plsc.VectorSubcoreMesh(core_axis_name="c", subcore_axis_name="s")

    @functools.partial(
        pl.kernel,
        mesh=mesh,
        compiler_params=pltpu.CompilerParams(
            use_tc_tiling_on_sc=True, needs_layout_passes=False
        ),
        out_type=jax.ShapeDtypeStruct((R4, 128), jnp.float32),
        scratch_types=[
            [pltpu.VMEM((D, 256), jnp.float32) for _ in range(2)],
            [pltpu.VMEM((64, 128), jnp.float32) for _ in range(2)],
            [pltpu.SemaphoreType.DMA for _ in range(2)],
            [pltpu.SemaphoreType.DMA for _ in range(2)],
        ],
    )
    def format_kernel(wt_hbm, tail_hbm, w4_hbm, S, O, lsem, wsem):
        wid = lax.axis_index("s") * NC + lax.axis_index("c")
        iota = lax.iota(jnp.int32, 16)
        nblk = base_cnt + jnp.where(wid < rem, 1, 0)
        start = wid * base_cnt + jnp.minimum(wid, rem)

        grow = [(iota + k) & 15 for k in range(16)]
        orow = iota // 4
        ocol = [32 * (iota % 4) + ((iota + k) & 15) for k in range(16)]

        def blk(j):
            return pl.multiple_of((start + j) * 256, 256)

        def start_load(j, buf):
            pltpu.async_copy(
                wt_hbm.at[:, pl.ds(blk(j), 256)], S[buf], lsem[buf]
            )

        def wait_load(buf):
            pltpu.make_async_copy(
                wt_hbm.at[:, pl.ds(0, 256)], S[buf], lsem[buf]
            ).wait()

        def start_write(j, buf):
            pltpu.async_copy(
                O[buf],
                w4_hbm.at[pl.ds(pl.multiple_of((start + j) * 64, 64), 64), :],
                wsem[buf],
            )

        def wait_write(buf):
            pltpu.make_async_copy(
                O[buf], w4_hbm.at[pl.ds(0, 64), :], wsem[buf]
            ).wait()

        def permute(buf):
            def pbody(m, carry):
                for half in range(2):
                    mm = 2 * m + half
                    h = mm // 16
                    dh16 = ((mm // 8) % 2) * 16
                    u0 = (mm % 8) * 16
                    vs = [
                        plsc.load_gather(
                            S[buf], [grow[k] + dh16, iota + (h * 128 + u0)]
                        )
                        for k in range(16)
                    ]
                    for k in range(16):
                        plsc.store_scatter(
                            O[buf],
                            [orow + (h * 32 + u0 // 4), ocol[k] + dh16],
                            vs[k],
                        )
                return carry

            lax.fori_loop(0, 16, pbody, 0)

        start_load(0, 0)

        def pair_body(p, carry):
            for buf in range(2):
                j = 2 * p + buf

                @pl.when(j < nblk)
                def _(j=j, buf=buf):
                    @pl.when(j + 1 < nblk)
                    def _():
                        start_load(j + 1, 1 - buf)

                    wait_load(buf)

                    @pl.when(j >= 2)
                    def _():
                        wait_write(buf)

                    permute(buf)
                    start_write(j, buf)

            return carry

        lax.fori_loop(0, (base_cnt + 2) // 2, pair_body, 0)
        wait_write(0)
        wait_write(1)

        @pl.when(wid == NW - 1)
        def _():
            pltpu.sync_copy(tail_hbm, S[0].at[pl.ds(0, 16), pl.ds(0, 128)])
            pltpu.sync_copy(
                S[0].at[pl.ds(0, 16), pl.ds(0, 128)],
                w4_hbm.at[pl.ds(R4 - 16, 16), :],
            )

    return format_kernel


@functools.cache
def _make_gather(V4, D, NB_ROWS, NB_COLS):
    info = plsc.get_sparse_core_info()
    NC, NS = info.num_cores, info.num_subcores
    NW = NC * NS
    assert NB_ROWS % NW == 0
    W = NB_ROWS // NW
    n_tasks = NB_COLS
    mesh = plsc.VectorSubcoreMesh(core_axis_name="c", subcore_axis_name="s")

    @functools.partial(
        pl.kernel,
        mesh=mesh,
        compiler_params=pltpu.CompilerParams(
            use_tc_tiling_on_sc=False, needs_layout_passes=False
        ),
        out_type=jax.ShapeDtypeStruct((NB_COLS, D, NB_ROWS), jnp.float32),
        scratch_types=[
            [pltpu.VMEM((W,), jnp.int32) for _ in range(2)],
            [pltpu.VMEM((W, D), jnp.float32) for _ in range(2)],
            [pltpu.VMEM((D, W), jnp.float32) for _ in range(2)],
            [pltpu.SemaphoreType.DMA for _ in range(2)],
            [pltpu.SemaphoreType.DMA for _ in range(2)],
        ],
    )
    def gather_kernel(table_hbm, idx_hbm, out_hbm, idxv, rows, outT, gsem, wsem):
        wid = lax.axis_index("s") * NC + lax.axis_index("c")
        b0 = wid * W
        iota = lax.iota(jnp.int32, 16)

        def start_gather(i, buf):
            pltpu.sync_copy(idx_hbm.at[pl.ds(i * NB_ROWS + b0, W)], idxv[buf])
            pltpu.async_copy(table_hbm.at[idxv[buf]], rows[buf], gsem[buf])

        def wait_gather(buf):
            pltpu.make_async_copy(
                table_hbm.at[idxv[buf]], rows[buf], gsem[buf]
            ).wait()

        def start_write(i, buf):
            pltpu.async_copy(
                outT[buf], out_hbm.at[i, :, pl.ds(b0, W)], wsem[buf]
            )

        def wait_write(buf):
            pltpu.make_async_copy(
                outT[buf], out_hbm.at[0, :, pl.ds(b0, W)], wsem[buf]
            ).wait()

        sdiag = [(iota + k) & 15 for k in range(16)]

        def transpose_task(buf):
            def body(jb, carry):
                j0 = jb * 16
                for db in range(D // 16):
                    d0 = db * 16
                    vs = [
                        plsc.load_gather(rows[buf], [iota + j0, sdiag[k] + d0])
                        for k in range(16)
                    ]
                    for k in range(16):
                        plsc.store_scatter(
                            outT[buf], [sdiag[k] + d0, iota + j0], vs[k]
                        )
                return carry

            lax.fori_loop(0, W // 16, body, 0)

        assert n_tasks % 2 == 0
        start_gather(0, 0)

        def pair_body(p, carry):
            for buf in range(2):
                i = 2 * p + buf

                @pl.when(i + 1 < n_tasks)
                def _():
                    start_gather(i + 1, 1 - buf)

                wait_gather(buf)

                @pl.when(i >= 2)
                def _():
                    wait_write(buf)

                transpose_task(buf)
                start_write(i, buf)
            return carry

        lax.fori_loop(0, n_tasks // 2, pair_body, 0)
        wait_write(0)
        wait_write(1)

    return gather_kernel


def kernel(X, weight):
    rows, cols = X.shape
    V, D = weight.shape
    flat_idx = X.T.reshape(rows * cols).astype(jnp.int32)
    tail = weight[V - 64 :].reshape(16, 128)
    w4 = _make_format(V, D)(weight.T, tail)
    out = _make_gather(V, D, rows, cols)(w4.reshape(V, D), flat_idx)
    return out.transpose(2, 0, 1)


# speedup vs baseline: 3.0578x; 1.0259x over previous
import functools

import jax
import jax.numpy as jnp
from jax import lax
from jax.experimental import pallas as pl
from jax.experimental.pallas import tpu as pltpu
from jax.experimental.pallas import tpu_sc as plsc


@functools.cache
def _make_format(V, D):
    info = plsc.get_sparse_core_info()
    NC, NS = info.num_cores, info.num_subcores
    NW = NC * NS
    TCOLS = V // 512
    R4 = V // 4
    base_cnt = TCOLS // NW
    rem = TCOLS % NW
    mesh = plsc.VectorSubcoreMesh(core_axis_name="c", subcore_axis_name="s")

    @functools.partial(
        pl.kernel,
        mesh=mesh,
        compiler_params=pltpu.CompilerParams(
            use_tc_tiling_on_sc=True, needs_layout_passes=False
        ),
        out_type=jax.ShapeDtypeStruct((R4, 128), jnp.float32),
        scratch_types=[
            [pltpu.VMEM((D, 512), jnp.float32) for _ in range(2)],
            [pltpu.VMEM((128, 128), jnp.float32) for _ in range(2)],
            [pltpu.SemaphoreType.DMA for _ in range(2)],
            [pltpu.SemaphoreType.DMA for _ in range(2)],
        ],
    )
    def format_kernel(wt_hbm, tail_hbm, w4_hbm, S, O, lsem, wsem):
        wid = lax.axis_index("s") * NC + lax.axis_index("c")
        iota = lax.iota(jnp.int32, 16)
        nblk = base_cnt + jnp.where(wid < rem, 1, 0)
        start = wid * base_cnt + jnp.minimum(wid, rem)

        grow = [(iota + k) & 15 for k in range(16)]
        orow = iota // 4
        ocol = [32 * (iota % 4) + ((iota + k) & 15) for k in range(16)]

        def blk(j):
            return pl.multiple_of((start + j) * 512, 512)

        def start_load(j, buf):
            pltpu.async_copy(
                wt_hbm.at[:, pl.ds(blk(j), 512)], S[buf], lsem[buf]
            )

        def wait_load(buf):
            pltpu.make_async_copy(
                wt_hbm.at[:, pl.ds(0, 512)], S[buf], lsem[buf]
            ).wait()

        def start_write(j, buf):
            pltpu.async_copy(
                O[buf],
                w4_hbm.at[
                    pl.ds(pl.multiple_of((start + j) * 128, 128), 128), :
                ],
                wsem[buf],
            )

        def wait_write(buf):
            pltpu.make_async_copy(
                O[buf], w4_hbm.at[pl.ds(0, 128), :], wsem[buf]
            ).wait()

        def permute(buf):
            def pbody(m, carry):
                for half in range(2):
                    mm = 2 * m + half
                    h = mm // 16
                    dh16 = ((mm // 8) % 2) * 16
                    u0 = (mm % 8) * 16
                    vs = [
                        plsc.load_gather(
                            S[buf], [grow[k] + dh16, iota + (h * 128 + u0)]
                        )
                        for k in range(16)
                    ]
                    for k in range(16):
                        plsc.store_scatter(
                            O[buf],
                            [orow + (h * 32 + u0 // 4), ocol[k] + dh16],
                            vs[k],
                        )
                return carry

            lax.fori_loop(0, 32, pbody, 0)

        start_load(0, 0)

        def pair_body(p, carry):
            for buf in range(2):
                j = 2 * p + buf

                @pl.when(j < nblk)
                def _(j=j, buf=buf):
                    @pl.when(j + 1 < nblk)
                    def _():
                        start_load(j + 1, 1 - buf)

                    wait_load(buf)

                    @pl.when(j >= 2)
                    def _():
                        wait_write(buf)

                    permute(buf)
                    start_write(j, buf)

            return carry

        lax.fori_loop(0, (base_cnt + 2) // 2, pair_body, 0)
        wait_write(0)
        wait_write(1)

        @pl.when(wid == NW - 1)
        def _():
            pltpu.sync_copy(tail_hbm, S[0].at[pl.ds(0, 16), pl.ds(0, 128)])
            pltpu.sync_copy(
                S[0].at[pl.ds(0, 16), pl.ds(0, 128)],
                w4_hbm.at[pl.ds(R4 - 16, 16), :],
            )

    return format_kernel


@functools.cache
def _make_gather(V4, D, NB_ROWS, NB_COLS):
    info = plsc.get_sparse_core_info()
    NC, NS = info.num_cores, info.num_subcores
    NW = NC * NS
    assert NB_ROWS % NW == 0
    W = NB_ROWS // NW
    n_tasks = NB_COLS
    mesh = plsc.VectorSubcoreMesh(core_axis_name="c", subcore_axis_name="s")

    @functools.partial(
        pl.kernel,
        mesh=mesh,
        compiler_params=pltpu.CompilerParams(
            use_tc_tiling_on_sc=False, needs_layout_passes=False
        ),
        out_type=jax.ShapeDtypeStruct((NB_COLS, D, NB_ROWS), jnp.float32),
        scratch_types=[
            [pltpu.VMEM((W,), jnp.int32) for _ in range(2)],
            [pltpu.VMEM((W, D), jnp.float32) for _ in range(2)],
            [pltpu.VMEM((D, W), jnp.float32) for _ in range(2)],
            [pltpu.SemaphoreType.DMA for _ in range(2)],
            [pltpu.SemaphoreType.DMA for _ in range(2)],
        ],
    )
    def gather_kernel(table_hbm, idx_hbm, out_hbm, idxv, rows, outT, gsem, wsem):
        wid = lax.axis_index("s") * NC + lax.axis_index("c")
        b0 = wid * W
        iota = lax.iota(jnp.int32, 16)

        def start_gather(i, buf):
            pltpu.sync_copy(idx_hbm.at[pl.ds(i * NB_ROWS + b0, W)], idxv[buf])
            pltpu.async_copy(table_hbm.at[idxv[buf]], rows[buf], gsem[buf])

        def wait_gather(buf):
            pltpu.make_async_copy(
                table_hbm.at[idxv[buf]], rows[buf], gsem[buf]
            ).wait()

        def start_write(i, buf):
            pltpu.async_copy(
                outT[buf], out_hbm.at[i, :, pl.ds(b0, W)], wsem[buf]
            )

        def wait_write(buf):
            pltpu.make_async_copy(
                outT[buf], out_hbm.at[0, :, pl.ds(b0, W)], wsem[buf]
            ).wait()

        sdiag = [(iota + k) & 15 for k in range(16)]

        def transpose_task(buf):
            def body(jb, carry):
                j0 = jb * 16
                for db in range(D // 16):
                    d0 = db * 16
                    vs = [
                        plsc.load_gather(rows[buf], [iota + j0, sdiag[k] + d0])
                        for k in range(16)
                    ]
                    for k in range(16):
                        plsc.store_scatter(
                            outT[buf], [sdiag[k] + d0, iota + j0], vs[k]
                        )
                return carry

            lax.fori_loop(0, W // 16, body, 0)

        assert n_tasks % 2 == 0
        start_gather(0, 0)

        def pair_body(p, carry):
            for buf in range(2):
                i = 2 * p + buf

                @pl.when(i + 1 < n_tasks)
                def _():
                    start_gather(i + 1, 1 - buf)

                wait_gather(buf)

                @pl.when(i >= 2)
                def _():
                    wait_write(buf)

                transpose_task(buf)
                start_write(i, buf)
            return carry

        lax.fori_loop(0, n_tasks // 2, pair_body, 0)
        wait_write(0)
        wait_write(1)

    return gather_kernel


def kernel(X, weight):
    rows, cols = X.shape
    V, D = weight.shape
    flat_idx = X.T.reshape(rows * cols).astype(jnp.int32)
    tail = weight[V - 64 :].reshape(16, 128)
    w4 = _make_format(V, D)(weight.T, tail)
    out = _make_gather(V, D, rows, cols)(w4.reshape(V, D), flat_idx)
    return out.transpose(2, 0, 1)
